# Initial kernel scaffold; baseline (speedup 1.0000x reference)
import jax
import jax.numpy as jnp
from jax.experimental import pallas as pl


def kernel(x, Wg, bg, We, be):
    raise NotImplementedError("write your pallas kernel here")


# speedup vs baseline: 1.8366x; 1.8366x over previous
import jax
import jax.numpy as jnp
from jax.experimental import pallas as pl
from jax.experimental.pallas import tpu as pltpu


def _moe_body(x_ref, Wg_ref, bg_ref, We_ref, be_ref, out_ref, probs_ref, xb_ref):
    e = pl.program_id(0)
    n_exp = pl.num_programs(0)

    @pl.when(e == 0)
    def _init():
        logits = (
            jnp.dot(x_ref[...], Wg_ref[...], preferred_element_type=jnp.float32)
            + bg_ref[...]
        )
        m = jnp.max(logits, axis=-1, keepdims=True)
        p = jnp.exp(logits - m)
        probs_ref[...] = p / jnp.sum(p, axis=-1, keepdims=True)
        xb_ref[...] = x_ref[...].astype(jnp.bfloat16)

    w = We_ref[0].astype(jnp.bfloat16)
    h = jnp.dot(xb_ref[...], w, preferred_element_type=jnp.float32)
    h = jnp.maximum(h + be_ref[0, 0], 0.0)

    lane = jax.lax.broadcasted_iota(jnp.int32, probs_ref.shape, 1)
    p_col = jnp.sum(
        jnp.where(lane == e, probs_ref[...], 0.0), axis=1, keepdims=True
    )
    contrib = h * p_col

    @pl.when(e == 0)
    def _first():
        out_ref[...] = contrib

    @pl.when(e > 0)
    def _rest():
        out_ref[...] += contrib

    del n_exp


def kernel(x, Wg, bg, We, be):
    T, H = x.shape
    E = We.shape[0]
    bg2 = bg.reshape(1, E)
    be3 = be.reshape(E, 1, H)
    return pl.pallas_call(
        _moe_body,
        grid=(E,),
        in_specs=[
            pl.BlockSpec((T, H), lambda e: (0, 0)),
            pl.BlockSpec((H, E), lambda e: (0, 0)),
            pl.BlockSpec((1, E), lambda e: (0, 0)),
            pl.BlockSpec((1, H, H), lambda e: (e, 0, 0)),
            pl.BlockSpec((1, 1, H), lambda e: (e, 0, 0)),
        ],
        out_specs=pl.BlockSpec((T, H), lambda e: (0, 0)),
        out_shape=jax.ShapeDtypeStruct((T, H), jnp.float32),
        scratch_shapes=[
            pltpu.VMEM((T, E), jnp.float32),
            pltpu.VMEM((T, H), jnp.bfloat16),
        ],
        compiler_params=pltpu.CompilerParams(
            dimension_semantics=("arbitrary",),
        ),
    )(x, Wg, bg2, We, be3)


# speedup vs baseline: 1.8398x; 1.0018x over previous
import jax
import jax.numpy as jnp
from jax.experimental import pallas as pl
from jax.experimental.pallas import tpu as pltpu


def _moe_body(x_ref, Wg_ref, bg_ref, We_ref, be_ref, out_ref, probs_ref):
    e = pl.program_id(0)

    @pl.when(e == 0)
    def _init():
        logits = (
            jnp.dot(x_ref[...], Wg_ref[...], preferred_element_type=jnp.float32)
            + bg_ref[...]
        )
        m = jnp.max(logits, axis=-1, keepdims=True)
        p = jnp.exp(logits - m)
        probs_ref[...] = p / jnp.sum(p, axis=-1, keepdims=True)

    h = jnp.dot(x_ref[...], We_ref[0], preferred_element_type=jnp.float32)
    h = jnp.maximum(h + be_ref[0, 0], 0.0)

    lane = jax.lax.broadcasted_iota(jnp.int32, probs_ref.shape, 1)
    p_col = jnp.sum(
        jnp.where(lane == e, probs_ref[...], 0.0), axis=1, keepdims=True
    )
    contrib = h * p_col

    @pl.when(e == 0)
    def _first():
        out_ref[...] = contrib

    @pl.when(e > 0)
    def _rest():
        out_ref[...] += contrib


def kernel(x, Wg, bg, We, be):
    T, H = x.shape
    E = We.shape[0]
    bg2 = bg.reshape(1, E)
    be3 = be.reshape(E, 1, H)
    return pl.pallas_call(
        _moe_body,
        grid=(E,),
        in_specs=[
            pl.BlockSpec((T, H), lambda e: (0, 0)),
            pl.BlockSpec((H, E), lambda e: (0, 0)),
            pl.BlockSpec((1, E), lambda e: (0, 0)),
            pl.BlockSpec((1, H, H), lambda e: (e, 0, 0)),
            pl.BlockSpec((1, 1, H), lambda e: (e, 0, 0)),
        ],
        out_specs=pl.BlockSpec((T, H), lambda e: (0, 0)),
        out_shape=jax.ShapeDtypeStruct((T, H), jnp.float32),
        scratch_shapes=[
            pltpu.VMEM((T, E), jnp.float32),
        ],
        compiler_params=pltpu.CompilerParams(
            dimension_semantics=("arbitrary",),
        ),
    )(x, Wg, bg2, We, be3)
